# Initial kernel scaffold; baseline (speedup 1.0000x reference)
import functools

import jax
import jax.numpy as jnp
from jax import lax
from jax.experimental import pallas as pl
from jax.experimental.pallas import tpu as pltpu
from jax.experimental.pallas import tpu_sc as plsc


_INFO = plsc.get_sparse_core_info()
_NC = _INFO.num_cores
_NS = _INFO.num_subcores
_NW = _NC * _NS


@functools.lru_cache(maxsize=None)
def _make_gather(N: int, D: int, CK: int):
    b_per_w = N // _NW
    nchunk = b_per_w // CK
    npair = nchunk // 2
    mesh = plsc.VectorSubcoreMesh(core_axis_name="c", subcore_axis_name="s")

    @functools.partial(
        pl.kernel,
        mesh=mesh,
        out_type=jax.ShapeDtypeStruct((N, D), jnp.float32),
        scratch_types=[
            pltpu.VMEM((nchunk, CK), jnp.int32),
            pltpu.VMEM((CK, D), jnp.float32),
            pltpu.VMEM((CK, D), jnp.float32),
            pltpu.SemaphoreType.DMA,
            pltpu.SemaphoreType.DMA,
        ],
    )
    def gather_kernel(table_hbm, idx_hbm, out_hbm, idx_v, buf0, buf1, sem0, sem1):
        wid = lax.axis_index("s") * _NC + lax.axis_index("c")
        base = wid * b_per_w
        pltpu.sync_copy(idx_hbm.at[wid], idx_v)

        def body(i, _):
            g0 = 2 * i
            g1 = g0 + 1
            cp0 = pltpu.async_copy(table_hbm.at[idx_v.at[g0]], buf0, sem0)
            cp1 = pltpu.async_copy(table_hbm.at[idx_v.at[g1]], buf1, sem1)
            cp0.wait()
            pltpu.sync_copy(buf0, out_hbm.at[pl.ds(base + g0 * CK, CK)])
            cp1.wait()
            pltpu.sync_copy(buf1, out_hbm.at[pl.ds(base + g1 * CK, CK)])
            return 0

        lax.fori_loop(0, npair, body, 0)

    return gather_kernel


def kernel(token_indices, table):
    B, T = token_indices.shape
    V, D = table.shape
    N = B * T
    CK = 4
    idx = token_indices.astype(jnp.int32).reshape(_NW, (N // _NW) // CK, CK)
    out = _make_gather(N, D, CK)(table, idx)
    return out.reshape(B, T, D)


# speedup vs baseline: 1.9323x; 1.9323x over previous
import functools

import jax
import jax.numpy as jnp
from jax import lax
from jax.experimental import pallas as pl
from jax.experimental.pallas import tpu as pltpu
from jax.experimental.pallas import tpu_sc as plsc


_INFO = plsc.get_sparse_core_info()
_NC = _INFO.num_cores
_NS = _INFO.num_subcores
_NW = _NC * _NS


@functools.lru_cache(maxsize=None)
def _make_gather(N: int, D: int, CK: int):
    b_per_w = N // _NW
    nchunk = b_per_w // CK
    npair = nchunk // 2
    mesh = plsc.VectorSubcoreMesh(core_axis_name="c", subcore_axis_name="s")

    @functools.partial(
        pl.kernel,
        mesh=mesh,
        out_type=jax.ShapeDtypeStruct((N, D), jnp.float32),
        scratch_types=[
            pltpu.VMEM((nchunk, CK), jnp.int32),
            pltpu.VMEM((CK, D), jnp.float32),
            pltpu.VMEM((CK, D), jnp.float32),
            pltpu.SemaphoreType.DMA,
            pltpu.SemaphoreType.DMA,
        ],
    )
    def gather_kernel(table_hbm, idx_hbm, out_hbm, idx_v, buf0, buf1, sem0, sem1):
        wid = lax.axis_index("s") * _NC + lax.axis_index("c")
        base = wid * b_per_w
        pltpu.sync_copy(idx_hbm.at[wid], idx_v)

        pltpu.async_copy(table_hbm.at[idx_v.at[0]], buf0, sem0)

        def body(i, _):
            g0 = 2 * i
            g1 = g0 + 1
            pltpu.make_async_copy(table_hbm.at[idx_v.at[g0]], buf0, sem0).wait()
            pltpu.async_copy(table_hbm.at[idx_v.at[g1]], buf1, sem1)
            pltpu.sync_copy(buf0, out_hbm.at[pl.ds(base + g0 * CK, CK)])
            pltpu.make_async_copy(table_hbm.at[idx_v.at[g1]], buf1, sem1).wait()
            nxt = (g1 + 1) % nchunk
            pltpu.async_copy(table_hbm.at[idx_v.at[nxt]], buf0, sem0)
            pltpu.sync_copy(buf1, out_hbm.at[pl.ds(base + g1 * CK, CK)])
            return 0

        lax.fori_loop(0, npair, body, 0)
        pltpu.make_async_copy(table_hbm.at[idx_v.at[0]], buf0, sem0).wait()

    return gather_kernel


def kernel(token_indices, table):
    B, T = token_indices.shape
    V, D = table.shape
    N = B * T
    CK = 4
    idx = token_indices.astype(jnp.int32).reshape(_NW, (N // _NW) // CK, CK)
    out = _make_gather(N, D, CK)(table, idx)
    return out.reshape(B, T, D)
